# baseline (speedup 1.0000x reference)
import jax
import jax.numpy as jnp
from jax.experimental import pallas as pl

_NEG = -100000000.0
_NHEADS = 2


def kernel(TX, X, MX, MY, params):
    f32 = jnp.float32
    Bn, Tn, Cn = X.shape
    L = params["chan_init"]["w"].shape[1]
    E = Tn * Cn
    dk = L // _NHEADS
    scale = 1.0 / (dk ** 0.5)
    H = _NHEADS
    n_layers = len(params["layers"])

    txc = TX[:, :, None]

    tw = params["time_init"]["w"]
    tb = params["time_init"]["b"][None, :]
    cw = params["chan_init"]["w"]
    cb = params["chan_init"]["b"][None, :]
    ew0 = params["edge_init"]["w"][0:1]
    ew1 = params["edge_init"]["w"][1:2]
    eb = params["edge_init"]["b"][None, :]

    layer_ws = []
    for lp in params["layers"]:
        layer_ws += [lp["attn"]["k"]["w"], lp["attn"]["k"]["b"][None, :],
                     lp["attn"]["v"]["w"], lp["attn"]["v"]["b"][None, :],
                     lp["edge_nn"]["w"], lp["edge_nn"]["b"][None, :],
                     lp["attn"]["q"]["w"], lp["attn"]["q"]["b"][None, :],
                     lp["attn"]["o"]["w"], lp["attn"]["o"]["b"][None, :]]

    def body(txc_ref, x_ref, mx_ref, my_ref,
             tw_ref, tb_ref, cw_ref, cb_ref, ew0_ref, ew1_ref, eb_ref,
             *rest):
      lw_refs = rest[:-1]
      out_ref = rest[-1]
      for b in range(Bn):
        x2 = x_ref[b]
        my2 = my_ref[b]
        mask2 = mx_ref[b] + my2
        txk = txc_ref[b]

        mask3 = mask2[:, :, None]
        tnz = (jnp.sum(mask2, axis=1, keepdims=True) > 0).astype(f32)
        cnz = (jnp.sum(mask2.T, axis=1, keepdims=True) > 0).astype(f32)

        d_i = jax.lax.broadcasted_iota(jnp.int32, (2 * L, 2 * H), 0)
        h_i = jax.lax.broadcasted_iota(jnp.int32, (2 * L, 2 * H), 1)
        S4 = (d_i // dk == h_i).astype(f32)
        d_j = jax.lax.broadcasted_iota(jnp.int32, (2 * H, 2 * L), 1)
        h_j = jax.lax.broadcasted_iota(jnp.int32, (2 * H, 2 * L), 0)
        ST4 = (d_j // dk == h_j).astype(f32)
        T_f = jnp.sin(txk * tw_ref[...] + tb_ref[...])
        C_f = jnp.maximum(cw_ref[...] + cb_ref[...], 0.0)

        U = jnp.maximum(x2[:, :, None] * ew0_ref[...][None]
                        + my2[:, :, None] * ew1_ref[...][None]
                        + eb_ref[...][None], 0.0).reshape(E, L)

        for li in range(n_layers):
            kw, kb, vw, vb, enw, enb, qw, qb, ow, ob = (
                r[...] for r in lw_refs[10 * li:10 * li + 10])
            z = jnp.zeros((L, L), f32)
            w0 = jnp.concatenate(
                [z, kw[:L], z, vw[:L], enw[L:2 * L], qw, z], axis=1)
            w1 = jnp.concatenate(
                [kw[:L], z, vw[:L], z, enw[2 * L:3 * L], z, qw], axis=1)
            w2 = jnp.concatenate(
                [kw[L:], kw[L:], vw[L:], vw[L:], enw[:L]], axis=1)
            bc = jnp.concatenate([kb, kb, vb, vb, enb, qb, qb], axis=1)

            TP = jnp.dot(T_f, w0, preferred_element_type=f32)
            CP = jnp.dot(C_f, w1, preferred_element_type=f32) + bc
            UP = jnp.dot(U, w2, preferred_element_type=f32)
            TC3 = TP[:, None, :] + CP[None, :, :]
            P3 = (UP.reshape(Tn, Cn, 5 * L) + TC3[:, :, 0:5 * L])

            KV = jnp.maximum(P3[:, :, 0:4 * L], 0.0)
            epre = P3[:, :, 4 * L:5 * L].reshape(E, L)
            Q2 = jnp.maximum(TC3[:, :, 5 * L:7 * L], 0.0)

            prod = (Q2 * KV[:, :, 0:2 * L]).reshape(E, 2 * L)
            s4 = (jnp.dot(prod, S4, preferred_element_type=f32)
                  .reshape(Tn, Cn, 2 * H)) * scale
            s4 = jnp.where(mask3 > 0.0, s4, _NEG)

            sT = s4[:, :, 0:H]
            eT = jnp.exp(sT - jnp.max(sT, axis=1, keepdims=True))
            awT = eT / jnp.sum(eT, axis=1, keepdims=True)
            sC = s4[:, :, H:2 * H]
            eC = jnp.exp(sC - jnp.max(sC, axis=0, keepdims=True))
            awC = eC / jnp.sum(eC, axis=0, keepdims=True)

            aw4 = jnp.concatenate([awT, awC], axis=2).reshape(E, 2 * H)
            awb = jnp.dot(aw4, ST4, preferred_element_type=f32)
            WV = (awb.reshape(Tn, Cn, 2 * L)) * KV[:, :, 2 * L:4 * L]
            avT = jnp.sum(WV[:, :, 0:L], axis=1)
            avC = jnp.sum(WV[:, :, L:2 * L], axis=0)

            T_new = (jnp.dot(avT, ow, preferred_element_type=f32) + ob) * tnz
            C_new = (jnp.dot(avC, ow, preferred_element_type=f32) + ob) * cnz

            U = jnp.maximum(U + epre, 0.0)
            T_f = T_new
            C_f = C_new

        out_ref[b] = (U.reshape(Tn, Cn, L) * mask3).reshape(E, L)

    data_specs = [
        pl.BlockSpec((Bn, Tn, 1), lambda: (0, 0, 0)),
        pl.BlockSpec((Bn, Tn, Cn), lambda: (0, 0, 0)),
        pl.BlockSpec((Bn, Tn, Cn), lambda: (0, 0, 0)),
        pl.BlockSpec((Bn, Tn, Cn), lambda: (0, 0, 0)),
    ]
    w_arrays = [tw, tb, cw, cb, ew0, ew1, eb] + layer_ws
    w_specs = [pl.BlockSpec(a.shape, lambda: (0, 0)) for a in w_arrays]

    out = pl.pallas_call(
        body,
        in_specs=data_specs + w_specs,
        out_specs=pl.BlockSpec((Bn, E, L), lambda: (0, 0, 0)),
        out_shape=jax.ShapeDtypeStruct((Bn, E, L), f32),
    )(txc, X, MX, MY, *w_arrays)
    return out.reshape(Bn, Tn, Cn, L)


# speedup vs baseline: 1.0841x; 1.0841x over previous
import jax
import jax.numpy as jnp
from jax.experimental import pallas as pl

_NEG = -100000000.0
_NHEADS = 2


def kernel(TX, X, MX, MY, params):
    f32 = jnp.float32
    Bn, Tn, Cn = X.shape
    L = params["chan_init"]["w"].shape[1]
    E = Tn * Cn
    dk = L // _NHEADS
    scale = 1.0 / (dk ** 0.5)
    H = _NHEADS
    n_layers = len(params["layers"])

    txc = TX[:, :, None]

    tw = params["time_init"]["w"]
    tb = params["time_init"]["b"][None, :]
    cw = params["chan_init"]["w"]
    cb = params["chan_init"]["b"][None, :]
    ew0 = params["edge_init"]["w"][0:1]
    ew1 = params["edge_init"]["w"][1:2]
    eb = params["edge_init"]["b"][None, :]

    layer_ws = []
    for lp in params["layers"]:
        layer_ws += [lp["attn"]["k"]["w"], lp["attn"]["k"]["b"][None, :],
                     lp["attn"]["v"]["w"], lp["attn"]["v"]["b"][None, :],
                     lp["edge_nn"]["w"], lp["edge_nn"]["b"][None, :],
                     lp["attn"]["q"]["w"], lp["attn"]["q"]["b"][None, :],
                     lp["attn"]["o"]["w"], lp["attn"]["o"]["b"][None, :]]

    def body(txc_ref, x_ref, mx_ref, my_ref,
             tw_ref, tb_ref, cw_ref, cb_ref, ew0_ref, ew1_ref, eb_ref,
             *rest):
        lw_refs = rest[:-1]
        out_ref = rest[-1]

        x2 = x_ref[0]
        my2 = my_ref[0]
        mask2 = mx_ref[0] + my2
        txk = txc_ref[0]

        mask3 = mask2[:, :, None]
        negm = (mask3 - 1.0) * (-_NEG)
        tnz = (jnp.sum(mask2, axis=1, keepdims=True) > 0).astype(f32)
        cnz = (jnp.sum(mask2.T, axis=1, keepdims=True) > 0).astype(f32)

        d_i = jax.lax.broadcasted_iota(jnp.int32, (2 * L, 2 * H), 0)
        h_i = jax.lax.broadcasted_iota(jnp.int32, (2 * L, 2 * H), 1)
        S4 = (d_i // dk == h_i).astype(f32) * scale
        d_j = jax.lax.broadcasted_iota(jnp.int32, (2 * H, 2 * L), 1)
        h_j = jax.lax.broadcasted_iota(jnp.int32, (2 * H, 2 * L), 0)
        ST4 = (d_j // dk == h_j).astype(f32)
        T_f = jnp.sin(txk * tw_ref[...] + tb_ref[...])
        C_f = jnp.maximum(cw_ref[...] + cb_ref[...], 0.0)

        U = jnp.maximum(x2[:, :, None] * ew0_ref[...][None]
                        + my2[:, :, None] * ew1_ref[...][None]
                        + eb_ref[...][None], 0.0).reshape(E, L)

        for li in range(n_layers):
            kw, kb, vw, vb, enw, enb, qw, qb, ow, ob = (
                r[...] for r in lw_refs[10 * li:10 * li + 10])
            z = jnp.zeros((L, L), f32)
            w0 = jnp.concatenate(
                [z, kw[:L], z, vw[:L], enw[L:2 * L], qw, z], axis=1)
            w1 = jnp.concatenate(
                [kw[:L], z, vw[:L], z, enw[2 * L:3 * L], z, qw], axis=1)
            w2 = jnp.concatenate(
                [kw[L:], kw[L:], vw[L:], vw[L:], enw[:L]], axis=1)
            bc = jnp.concatenate([kb, kb, vb, vb, enb, qb, qb], axis=1)

            TP = jnp.dot(T_f, w0, preferred_element_type=f32)
            CP = jnp.dot(C_f, w1, preferred_element_type=f32) + bc
            UP = jnp.dot(U, w2, preferred_element_type=f32)
            TC3 = TP[:, None, :] + CP[None, :, :]
            P3 = (UP.reshape(Tn, Cn, 5 * L) + TC3[:, :, 0:5 * L])

            KV = jnp.maximum(P3[:, :, 0:4 * L], 0.0)
            epre = P3[:, :, 4 * L:5 * L].reshape(E, L)
            Q2 = jnp.maximum(TC3[:, :, 5 * L:7 * L], 0.0)

            prod = (Q2 * KV[:, :, 0:2 * L]).reshape(E, 2 * L)
            s4 = (jnp.dot(prod, S4, preferred_element_type=f32)
                  .reshape(Tn, Cn, 2 * H)) + negm

            sT = s4[:, :, 0:H]
            eT = jnp.exp(sT - jnp.max(sT, axis=1, keepdims=True))
            awT = eT / jnp.sum(eT, axis=1, keepdims=True)
            sC = s4[:, :, H:2 * H]
            eC = jnp.exp(sC - jnp.max(sC, axis=0, keepdims=True))
            awC = eC / jnp.sum(eC, axis=0, keepdims=True)

            aw4 = jnp.concatenate([awT, awC], axis=2).reshape(E, 2 * H)
            awb = jnp.dot(aw4, ST4, preferred_element_type=f32)
            WV = (awb.reshape(Tn, Cn, 2 * L)) * KV[:, :, 2 * L:4 * L]
            avT = jnp.sum(WV[:, :, 0:L], axis=1)
            avC = jnp.sum(WV[:, :, L:2 * L], axis=0)

            T_new = (jnp.dot(avT, ow, preferred_element_type=f32) + ob) * tnz
            C_new = (jnp.dot(avC, ow, preferred_element_type=f32) + ob) * cnz

            U = jnp.maximum(U + epre, 0.0)
            T_f = T_new
            C_f = C_new

        out_ref[0] = (U.reshape(Tn, Cn, L) * mask3).reshape(E, L)

    data_specs = [
        pl.BlockSpec((1, Tn, 1), lambda b: (b, 0, 0)),
        pl.BlockSpec((1, Tn, Cn), lambda b: (b, 0, 0)),
        pl.BlockSpec((1, Tn, Cn), lambda b: (b, 0, 0)),
        pl.BlockSpec((1, Tn, Cn), lambda b: (b, 0, 0)),
    ]
    w_arrays = [tw, tb, cw, cb, ew0, ew1, eb] + layer_ws
    w_specs = [pl.BlockSpec(a.shape, lambda b: (0, 0)) for a in w_arrays]

    out = pl.pallas_call(
        body,
        grid=(Bn,),
        in_specs=data_specs + w_specs,
        out_specs=pl.BlockSpec((1, E, L), lambda b: (b, 0, 0)),
        out_shape=jax.ShapeDtypeStruct((Bn, E, L), f32),
    )(txc, X, MX, MY, *w_arrays)
    return out.reshape(Bn, Tn, Cn, L)
